# baseline (speedup 1.0000x reference)
import functools

import jax
import jax.numpy as jnp
from jax import lax
from jax.experimental import pallas as pl
from jax.experimental.pallas import tpu as pltpu
from jax.experimental.pallas import tpu_sc as plsc

N_FINE_TILE = 2592


def _relu(v):
    return jnp.maximum(v, 0.0)


def _lrelu(v):
    return jnp.where(v >= 0, v, 0.01 * v)


def _f32dot(a, b):
    return jnp.dot(a, b, preferred_element_type=jnp.float32)


def _prep_kernel(h3_ref, lea_ref, ei_ref,
                 dst1_w, dst1_b, dst2_w, dst2_b,
                 src2_w, src2_b, edg2_w, edg2_b,
                 msg1_w, msg1_b, msg2_w,
                 lat1_w, lat1_b, lat2_w, lat2_b,
                 lato_w, lato_b, lsk_w, lsk_b,
                 A_o, E2_o, M2_o, cb_o, hdp_o, hdst_o, enc_o,
                 dst3_o, dstc3_o, idx_o, *, n_tiles, tile, m_coarse):
    n = ei_ref.shape[1]
    for k in range(n_tiles):
        row = ei_ref[1, pl.ds(k * tile, tile)].reshape(1, tile)
        dst3_o[k] = row
        dstc3_o[k] = jnp.transpose(row)
    dstrow = ei_ref[1:2, :]
    idx_o[0:1, 0:n] = dstrow
    idx_o[0:1, n:2 * n] = dstrow + m_coarse
    ws = msg1_w[0:128, :]
    wd = msg1_w[128:256, :]
    we = msg1_w[256:384, :]
    A_o[...] = _f32dot(src2_w[...], ws).astype(jnp.bfloat16)
    E2_o[...] = _f32dot(edg2_w[...], we).astype(jnp.bfloat16)
    M2_o[...] = msg2_w[...].astype(jnp.bfloat16)
    cb_o[...] = (_f32dot(src2_b[...], ws) + _f32dot(edg2_b[...], we)
                 + msg1_b[...])
    hd1 = _relu(_f32dot(h3_ref[...], dst1_w[...]) + dst1_b[...])
    hdst = _f32dot(hd1, dst2_w[...]) + dst2_b[...]
    hdst_o[...] = hdst
    hdp_o[...] = _f32dot(hdst, wd).astype(jnp.bfloat16)
    lea = lea_ref[...]
    a1 = _lrelu(lea[:, 0:1] * lat1_w[0:1, :] + lea[:, 1:2] * lat1_w[1:2, :]
                + lat1_b[...])
    a2 = _lrelu(_f32dot(a1, lat2_w[...]) + lat2_b[...])
    enc_o[...] = (_f32dot(a2, lato_w[...]) + lato_b[...]
                  + lea[:, 0:1] * lsk_w[0:1, :] + lea[:, 1:2] * lsk_w[1:2, :]
                  + lsk_b[...])


def _repack_kernel(xv_ref, xstd_ref, *, bt, f):
    for b in range(bt):
        xstd_ref[b] = jnp.swapaxes(
            xv_ref[b * f:(b + 1) * f, 0, :], 0, 1).astype(jnp.bfloat16)


def _main_kernel(x_ref, ea_ref, dst_ref, dstc_ref,
                 src1_w, src1_b, edg1_w, edg1_b, msg2_b,
                 A_r, E2_r, M2_r, cb_r, hdp_r,
                 m_out, *, bt, m_coarse):
    tile = dst_ref.shape[2]
    dstc = dstc_ref[0]
    cols = lax.broadcasted_iota(jnp.int32, (tile, m_coarse), 1)
    oh = (cols == dstc).astype(jnp.bfloat16)

    g = _f32dot(oh, hdp_r[...])
    ea = ea_ref[...]
    e1 = _relu(ea[:, 0:1] * edg1_w[0:1, :] + ea[:, 1:2] * edg1_w[1:2, :]
               + edg1_b[...])
    epre = _f32dot(e1.astype(jnp.bfloat16), E2_r[...])
    base = epre + g + cb_r[...]
    w1 = src1_w[...].astype(jnp.bfloat16)
    for b in range(bt):
        h1 = _relu(_f32dot(x_ref[b], w1) + src1_b[...])
        hidden = _relu(_f32dot(h1.astype(jnp.bfloat16), A_r[...]) + base)
        m_out[b] = _f32dot(hidden.astype(jnp.bfloat16), M2_r[...]) + msg2_b[...]


def _sc_scatter_kernel(m_r, idx_r, z_r, zi_r, out_r, buf_v, idx_v, acc_sh,
                       *, ch, full_ch, nch_per_w, tail_rows):
    cid = lax.axis_index("c")
    sid = lax.axis_index("s")
    wid = sid * 2 + cid

    @pl.when(sid == 0)
    def _init():
        pltpu.sync_copy(z_r, acc_sh)
    plsc.subcore_barrier()

    def body(j, carry):
        g = j * 32 + wid

        @pl.when(g < full_ch)
        def _full():
            base = g * ch
            pltpu.sync_copy(m_r.at[pl.ds(base, ch)], buf_v)
            pltpu.sync_copy(idx_r.at[0, pl.ds(base, ch)], idx_v)
            pltpu.sync_copy(buf_v, acc_sh.at[idx_v], add=True)

        if tail_rows:
            @pl.when(g == full_ch)
            def _tail():
                base = g * ch
                pad = ch - tail_rows
                pltpu.sync_copy(m_r.at[pl.ds(base, tail_rows)],
                                buf_v.at[pl.ds(0, tail_rows)])
                pltpu.sync_copy(z_r.at[pl.ds(0, pad)],
                                buf_v.at[pl.ds(tail_rows, pad)])
                pltpu.sync_copy(idx_r.at[0, pl.ds(base, tail_rows)],
                                idx_v.at[pl.ds(0, tail_rows)])
                pltpu.sync_copy(zi_r.at[0, pl.ds(0, pad)],
                                idx_v.at[pl.ds(tail_rows, pad)])
                pltpu.sync_copy(buf_v, acc_sh.at[idx_v], add=True)
        return carry

    lax.fori_loop(0, nch_per_w, body, 0)
    plsc.subcore_barrier()

    @pl.when(sid == 0)
    def _flush():
        pltpu.sync_copy(acc_sh, out_r.at[cid])


def _finish_kernel(hdst_r, part_r, upd1_w, upd1_b, upd2_w, upd2_b, hc_out,
                   *, bt, m_coarse):
    u1h = upd1_w[0:128, :]
    u1a = upd1_w[128:256, :]
    hpre = _f32dot(hdst_r[...], u1h) + upd1_b[...]
    for b in range(bt):
        agg = (part_r[0, pl.ds(b * m_coarse, m_coarse), :]
               + part_r[1, pl.ds(b * m_coarse, m_coarse), :])
        u = _relu(hpre + _f32dot(agg, u1a))
        hc_out[pl.ds(b * m_coarse, m_coarse), 0, :] = (
            _f32dot(u, upd2_w[...]) + upd2_b[...])


def kernel(x, h3_nodes, edge_attr_f2c, latent_edge_attr, params,
           edge_index_f2c, latent_edge_index):
    b, t, n, f = x.shape
    bt = b * t
    m_coarse = h3_nodes.shape[0]
    n_lat = latent_edge_attr.shape[0]
    hid = params["src1"]["w"].shape[1]
    out = params["src2"]["w"].shape[1]
    eout = params["edg2"]["w"].shape[1]

    tile = N_FINE_TILE if n % N_FINE_TILE == 0 else max(
        d for d in range(8, 4097, 8) if n % d == 0)
    n_tiles = n // tile

    xv = jnp.transpose(x, (0, 1, 3, 2)).reshape(bt * f, 1, n)
    cn = 4096
    x2 = pl.pallas_call(
        functools.partial(_repack_kernel, bt=bt, f=f),
        grid=(pl.cdiv(n, cn),),
        in_specs=[pl.BlockSpec((bt * f, 1, cn), lambda i: (0, 0, i))],
        out_specs=pl.BlockSpec((bt, cn, f), lambda i: (0, i, 0)),
        out_shape=jax.ShapeDtypeStruct((bt, n, f), jnp.bfloat16),
    )(xv)

    def b2(v):
        return v.reshape(1, -1)

    p = params
    full = lambda a: pl.BlockSpec(a.shape, lambda i: (0,) * a.ndim)

    prep_args = (
        h3_nodes, latent_edge_attr, edge_index_f2c.astype(jnp.int32),
        p["dst1"]["w"], b2(p["dst1"]["b"]), p["dst2"]["w"], b2(p["dst2"]["b"]),
        p["src2"]["w"], b2(p["src2"]["b"]), p["edg2"]["w"], b2(p["edg2"]["b"]),
        p["msg1"]["w"], b2(p["msg1"]["b"]), p["msg2"]["w"],
        p["lat1"]["w"], b2(p["lat1"]["b"]), p["lat2"]["w"], b2(p["lat2"]["b"]),
        p["lato"]["w"], b2(p["lato"]["b"]),
        p["latskip"]["w"], b2(p["latskip"]["b"]),
    )
    prep_out = (
        jax.ShapeDtypeStruct((hid, 128), jnp.bfloat16),
        jax.ShapeDtypeStruct((hid, 128), jnp.bfloat16),
        jax.ShapeDtypeStruct((hid, 128), jnp.bfloat16),
        jax.ShapeDtypeStruct((1, 128), jnp.float32),
        jax.ShapeDtypeStruct((m_coarse, 128), jnp.bfloat16),
        jax.ShapeDtypeStruct((m_coarse, out), jnp.float32),
        jax.ShapeDtypeStruct((n_lat, eout), jnp.float32),
        jax.ShapeDtypeStruct((n_tiles, 1, tile), jnp.int32),
        jax.ShapeDtypeStruct((n_tiles, tile, 1), jnp.int32),
        jax.ShapeDtypeStruct((1, bt * n), jnp.int32),
    )
    (A16, E216, M216, cb, hdp16, hdst, enc, dst, dstc, idxf) = pl.pallas_call(
        functools.partial(_prep_kernel, n_tiles=n_tiles, tile=tile,
                          m_coarse=m_coarse),
        out_shape=prep_out,
    )(*prep_args)

    main_args = (
        x2, edge_attr_f2c, dst, dstc,
        p["src1"]["w"], b2(p["src1"]["b"]),
        p["edg1"]["w"], b2(p["edg1"]["b"]), b2(p["msg2"]["b"]),
        A16, E216, M216, cb, hdp16,
    )
    main_in_specs = [
        pl.BlockSpec((bt, tile, f), lambda i: (0, i, 0)),
        pl.BlockSpec((tile, 2), lambda i: (i, 0)),
        pl.BlockSpec((1, 1, tile), lambda i: (i, 0, 0)),
        pl.BlockSpec((1, tile, 1), lambda i: (i, 0, 0)),
    ] + [full(a) for a in main_args[4:]]
    m_msgs = pl.pallas_call(
        functools.partial(_main_kernel, bt=bt, m_coarse=m_coarse),
        grid=(n_tiles,),
        in_specs=main_in_specs,
        out_specs=pl.BlockSpec((bt, tile, out), lambda i: (0, i, 0)),
        out_shape=jax.ShapeDtypeStruct((bt, n, out), jnp.float32),
    )(*main_args)
    m_flat = m_msgs.reshape(bt * n, out)

    n_rows = bt * n
    ch = 128
    full_ch = n_rows // ch
    tail_rows = n_rows - full_ch * ch
    nch_per_w = (full_ch + (1 if tail_rows else 0) + 31) // 32
    zeros = jnp.zeros((bt * m_coarse, out), jnp.float32)
    zeros_i = jnp.zeros((1, 128), jnp.int32)
    mesh = plsc.VectorSubcoreMesh(core_axis_name="c", subcore_axis_name="s")
    parts = pl.kernel(
        functools.partial(_sc_scatter_kernel, ch=ch, full_ch=full_ch,
                          nch_per_w=nch_per_w, tail_rows=tail_rows),
        mesh=mesh,
        out_type=jax.ShapeDtypeStruct((2, bt * m_coarse, out), jnp.float32),
        scratch_types=[
            pltpu.VMEM((ch, out), jnp.float32),
            pltpu.VMEM((ch,), jnp.int32),
            pltpu.VMEM_SHARED((bt * m_coarse, out), jnp.float32),
        ],
    )(m_flat, idxf, zeros, zeros_i)

    hc = pl.pallas_call(
        functools.partial(_finish_kernel, bt=bt, m_coarse=m_coarse),
        out_shape=jax.ShapeDtypeStruct((bt * m_coarse, 1, out), jnp.float32),
    )(hdst, parts, p["upd1"]["w"], b2(p["upd1"]["b"]),
      p["upd2"]["w"], b2(p["upd2"]["b"]))

    return hc.reshape(b, t, m_coarse, out), latent_edge_index, enc


# speedup vs baseline: 1.2151x; 1.2151x over previous
import functools

import jax
import jax.numpy as jnp
from jax import lax
from jax.experimental import pallas as pl
from jax.experimental.pallas import tpu as pltpu
from jax.experimental.pallas import tpu_sc as plsc

N_FINE_TILE = 2592


def _relu(v):
    return jnp.maximum(v, 0.0)


def _lrelu(v):
    return jnp.where(v >= 0, v, 0.01 * v)


def _f32dot(a, b):
    return jnp.dot(a, b, preferred_element_type=jnp.float32)


def _prep_kernel(h3_ref, lea_ref, ei_ref,
                 dst1_w, dst1_b, dst2_w, dst2_b,
                 src2_w, src2_b, edg2_w, edg2_b,
                 msg1_w, msg1_b, msg2_w,
                 lat1_w, lat1_b, lat2_w, lat2_b,
                 lato_w, lato_b, lsk_w, lsk_b,
                 A_o, E2_o, M2_o, cb_o, hdp_o, hdst_o, enc_o,
                 dst3_o, dstc3_o, idx_o, *, n_tiles, tile, m_coarse):
    for k in range(n_tiles):
        row = ei_ref[1, pl.ds(k * tile, tile)].reshape(1, tile)
        dst3_o[k] = row
        dstc3_o[k] = jnp.transpose(row)
    n_sc = idx_o.shape[1] // 2
    dstrow = ei_ref[1:2, pl.ds(0, n_sc)]
    idx_o[0:1, 0:n_sc] = dstrow
    idx_o[0:1, n_sc:2 * n_sc] = dstrow + m_coarse
    ws = msg1_w[0:128, :]
    wd = msg1_w[128:256, :]
    we = msg1_w[256:384, :]
    A_o[...] = _f32dot(src2_w[...], ws).astype(jnp.bfloat16)
    E2_o[...] = _f32dot(edg2_w[...], we).astype(jnp.bfloat16)
    M2_o[...] = msg2_w[...].astype(jnp.bfloat16)
    cb_o[...] = (_f32dot(src2_b[...], ws) + _f32dot(edg2_b[...], we)
                 + msg1_b[...])
    hd1 = _relu(_f32dot(h3_ref[...], dst1_w[...]) + dst1_b[...])
    hdst = _f32dot(hd1, dst2_w[...]) + dst2_b[...]
    hdst_o[...] = hdst
    hdp_o[...] = _f32dot(hdst, wd).astype(jnp.bfloat16)
    lea = lea_ref[...]
    a1 = _lrelu(lea[:, 0:1] * lat1_w[0:1, :] + lea[:, 1:2] * lat1_w[1:2, :]
                + lat1_b[...])
    a2 = _lrelu(_f32dot(a1, lat2_w[...]) + lat2_b[...])
    enc_o[...] = (_f32dot(a2, lato_w[...]) + lato_b[...]
                  + lea[:, 0:1] * lsk_w[0:1, :] + lea[:, 1:2] * lsk_w[1:2, :]
                  + lsk_b[...])


def _repack_kernel(xv_ref, xstd_ref, *, bt, f):
    for b in range(bt):
        xstd_ref[b] = jnp.swapaxes(
            xv_ref[b * f:(b + 1) * f, 0, :], 0, 1).astype(jnp.bfloat16)


def _main_a_kernel(x_ref, ea_ref, dst_ref, dstc_ref,
                   src1_w, src1_b, edg1_w, edg1_b, msg2_b,
                   A_r, E2_r, M2_r, cb_r, hdp_r,
                   m_out, *, bt, m_coarse):
    tile = dst_ref.shape[2]
    dstc = dstc_ref[0]
    cols = lax.broadcasted_iota(jnp.int32, (tile, m_coarse), 1)
    oh = (cols == dstc).astype(jnp.bfloat16)

    g = _f32dot(oh, hdp_r[...])
    ea = ea_ref[...]
    e1 = _relu(ea[:, 0:1] * edg1_w[0:1, :] + ea[:, 1:2] * edg1_w[1:2, :]
               + edg1_b[...])
    epre = _f32dot(e1.astype(jnp.bfloat16), E2_r[...])
    base = epre + g + cb_r[...]
    w1 = src1_w[...].astype(jnp.bfloat16)
    for b in range(bt):
        h1 = _relu(_f32dot(x_ref[b], w1) + src1_b[...])
        hidden = _relu(_f32dot(h1.astype(jnp.bfloat16), A_r[...]) + base)
        m_out[b] = _f32dot(hidden.astype(jnp.bfloat16), M2_r[...]) + msg2_b[...]


def _main_b_kernel(x_ref, ea_ref, dst_ref, dstc_ref,
                   src1_w, src1_b, edg1_w, edg1_b, msg2_b,
                   A_r, E2_r, M2_r, cb_r, hdp_r,
                   agg_out, agg_s, *, bt, m_coarse):
    i = pl.program_id(0)

    @pl.when(i == 0)
    def _init():
        agg_s[...] = jnp.zeros_like(agg_s)

    tile = dst_ref.shape[2]
    dstt = dst_ref[0]
    rows = lax.broadcasted_iota(jnp.int32, (m_coarse, tile), 0)
    ohT = (rows == dstt).astype(jnp.bfloat16)
    dstc = dstc_ref[0]
    cols = lax.broadcasted_iota(jnp.int32, (tile, m_coarse), 1)
    oh = (cols == dstc).astype(jnp.bfloat16)

    g = _f32dot(oh, hdp_r[...])
    ea = ea_ref[...]
    e1 = _relu(ea[:, 0:1] * edg1_w[0:1, :] + ea[:, 1:2] * edg1_w[1:2, :]
               + edg1_b[...])
    epre = _f32dot(e1.astype(jnp.bfloat16), E2_r[...])
    base = epre + g + cb_r[...]
    w1 = src1_w[...].astype(jnp.bfloat16)
    for b in range(bt):
        h1 = _relu(_f32dot(x_ref[b], w1) + src1_b[...])
        hidden = _relu(_f32dot(h1.astype(jnp.bfloat16), A_r[...]) + base)
        m = (_f32dot(hidden.astype(jnp.bfloat16), M2_r[...])
             + msg2_b[...]).astype(jnp.bfloat16)
        agg_s[b] += _f32dot(ohT, m)

    @pl.when(i == pl.num_programs(0) - 1)
    def _flush():
        agg_out[...] = agg_s[...]


def _sc_scatter_kernel(m_r, idx_r, z_r, zi_r, out_r, buf_v, idx_v, acc_sh,
                       *, ch, full_ch, nch_per_w, tail_rows):
    cid = lax.axis_index("c")
    sid = lax.axis_index("s")
    wid = sid * 2 + cid

    @pl.when(sid == 0)
    def _init():
        pltpu.sync_copy(z_r, acc_sh)
    plsc.subcore_barrier()

    def body(j, carry):
        g = j * 32 + wid

        @pl.when(g < full_ch)
        def _full():
            base = g * ch
            pltpu.sync_copy(m_r.at[pl.ds(base, ch)], buf_v)
            pltpu.sync_copy(idx_r.at[0, pl.ds(base, ch)], idx_v)
            pltpu.sync_copy(buf_v, acc_sh.at[idx_v], add=True)

        if tail_rows:
            @pl.when(g == full_ch)
            def _tail():
                base = g * ch
                pad = ch - tail_rows
                pltpu.sync_copy(m_r.at[pl.ds(base, tail_rows)],
                                buf_v.at[pl.ds(0, tail_rows)])
                pltpu.sync_copy(z_r.at[pl.ds(0, pad)],
                                buf_v.at[pl.ds(tail_rows, pad)])
                pltpu.sync_copy(idx_r.at[0, pl.ds(base, tail_rows)],
                                idx_v.at[pl.ds(0, tail_rows)])
                pltpu.sync_copy(zi_r.at[0, pl.ds(0, pad)],
                                idx_v.at[pl.ds(tail_rows, pad)])
                pltpu.sync_copy(buf_v, acc_sh.at[idx_v], add=True)
        return carry

    lax.fori_loop(0, nch_per_w, body, 0)
    plsc.subcore_barrier()

    @pl.when(sid == 0)
    def _flush():
        pltpu.sync_copy(acc_sh, out_r.at[cid])


def _finish_kernel(hdst_r, part_r, aggb_r, upd1_w, upd1_b, upd2_w, upd2_b,
                   hc_out, *, bt, m_coarse):
    u1h = upd1_w[0:128, :]
    u1a = upd1_w[128:256, :]
    hpre = _f32dot(hdst_r[...], u1h) + upd1_b[...]
    for b in range(bt):
        agg = (part_r[0, pl.ds(b * m_coarse, m_coarse), :]
               + part_r[1, pl.ds(b * m_coarse, m_coarse), :]
               + aggb_r[b])
        u = _relu(hpre + _f32dot(agg, u1a))
        hc_out[pl.ds(b * m_coarse, m_coarse), 0, :] = (
            _f32dot(u, upd2_w[...]) + upd2_b[...])


def kernel(x, h3_nodes, edge_attr_f2c, latent_edge_attr, params,
           edge_index_f2c, latent_edge_index):
    b, t, n, f = x.shape
    bt = b * t
    m_coarse = h3_nodes.shape[0]
    n_lat = latent_edge_attr.shape[0]
    hid = params["src1"]["w"].shape[1]
    out = params["src2"]["w"].shape[1]
    eout = params["edg2"]["w"].shape[1]

    tile = N_FINE_TILE if n % N_FINE_TILE == 0 else max(
        d for d in range(8, 4097, 8) if n % d == 0)
    n_tiles = n // tile
    K_SC = max(1, (2 * n_tiles + 1) // 3)

    xv = jnp.transpose(x, (0, 1, 3, 2)).reshape(bt * f, 1, n)
    cn = 4096
    x2 = pl.pallas_call(
        functools.partial(_repack_kernel, bt=bt, f=f),
        grid=(pl.cdiv(n, cn),),
        in_specs=[pl.BlockSpec((bt * f, 1, cn), lambda i: (0, 0, i))],
        out_specs=pl.BlockSpec((bt, cn, f), lambda i: (0, i, 0)),
        out_shape=jax.ShapeDtypeStruct((bt, n, f), jnp.bfloat16),
    )(xv)

    def b2(v):
        return v.reshape(1, -1)

    p = params
    full = lambda a: pl.BlockSpec(a.shape, lambda i: (0,) * a.ndim)

    prep_args = (
        h3_nodes, latent_edge_attr, edge_index_f2c.astype(jnp.int32),
        p["dst1"]["w"], b2(p["dst1"]["b"]), p["dst2"]["w"], b2(p["dst2"]["b"]),
        p["src2"]["w"], b2(p["src2"]["b"]), p["edg2"]["w"], b2(p["edg2"]["b"]),
        p["msg1"]["w"], b2(p["msg1"]["b"]), p["msg2"]["w"],
        p["lat1"]["w"], b2(p["lat1"]["b"]), p["lat2"]["w"], b2(p["lat2"]["b"]),
        p["lato"]["w"], b2(p["lato"]["b"]),
        p["latskip"]["w"], b2(p["latskip"]["b"]),
    )
    prep_out = (
        jax.ShapeDtypeStruct((hid, 128), jnp.bfloat16),
        jax.ShapeDtypeStruct((hid, 128), jnp.bfloat16),
        jax.ShapeDtypeStruct((hid, 128), jnp.bfloat16),
        jax.ShapeDtypeStruct((1, 128), jnp.float32),
        jax.ShapeDtypeStruct((m_coarse, 128), jnp.bfloat16),
        jax.ShapeDtypeStruct((m_coarse, out), jnp.float32),
        jax.ShapeDtypeStruct((n_lat, eout), jnp.float32),
        jax.ShapeDtypeStruct((n_tiles, 1, tile), jnp.int32),
        jax.ShapeDtypeStruct((n_tiles, tile, 1), jnp.int32),
        jax.ShapeDtypeStruct((1, bt * K_SC * tile), jnp.int32),
    )
    (A16, E216, M216, cb, hdp16, hdst, enc, dst, dstc, idxf) = pl.pallas_call(
        functools.partial(_prep_kernel, n_tiles=n_tiles, tile=tile,
                          m_coarse=m_coarse),
        out_shape=prep_out,
    )(*prep_args)

    main_args = (
        x2, edge_attr_f2c, dst, dstc,
        p["src1"]["w"], b2(p["src1"]["b"]),
        p["edg1"]["w"], b2(p["edg1"]["b"]), b2(p["msg2"]["b"]),
        A16, E216, M216, cb, hdp16,
    )
    main_in_specs = [
        pl.BlockSpec((bt, tile, f), lambda i: (0, i, 0)),
        pl.BlockSpec((tile, 2), lambda i: (i, 0)),
        pl.BlockSpec((1, 1, tile), lambda i: (i, 0, 0)),
        pl.BlockSpec((1, tile, 1), lambda i: (i, 0, 0)),
    ] + [full(a) for a in main_args[4:]]
    m_msgs = pl.pallas_call(
        functools.partial(_main_a_kernel, bt=bt, m_coarse=m_coarse),
        grid=(K_SC,),
        in_specs=main_in_specs,
        out_specs=pl.BlockSpec((bt, tile, out), lambda i: (0, i, 0)),
        out_shape=jax.ShapeDtypeStruct((bt, K_SC * tile, out), jnp.float32),
    )(*main_args)
    m_flat = m_msgs.reshape(bt * K_SC * tile, out)

    n_rows = bt * K_SC * tile
    ch = 128
    full_ch = n_rows // ch
    tail_rows = n_rows - full_ch * ch
    nch_per_w = (full_ch + (1 if tail_rows else 0) + 31) // 32
    zeros = jnp.zeros((bt * m_coarse, out), jnp.float32)
    zeros_i = jnp.zeros((1, 128), jnp.int32)
    mesh = plsc.VectorSubcoreMesh(core_axis_name="c", subcore_axis_name="s")
    parts = pl.kernel(
        functools.partial(_sc_scatter_kernel, ch=ch, full_ch=full_ch,
                          nch_per_w=nch_per_w, tail_rows=tail_rows),
        mesh=mesh,
        out_type=jax.ShapeDtypeStruct((2, bt * m_coarse, out), jnp.float32),
        scratch_types=[
            pltpu.VMEM((ch, out), jnp.float32),
            pltpu.VMEM((ch,), jnp.int32),
            pltpu.VMEM_SHARED((bt * m_coarse, out), jnp.float32),
        ],
    )(m_flat, idxf, zeros, zeros_i)

    nb = n_tiles - K_SC
    main_b_in_specs = [
        pl.BlockSpec((bt, tile, f), lambda i: (0, i + K_SC, 0)),
        pl.BlockSpec((tile, 2), lambda i: (i + K_SC, 0)),
        pl.BlockSpec((1, 1, tile), lambda i: (i + K_SC, 0, 0)),
        pl.BlockSpec((1, tile, 1), lambda i: (i + K_SC, 0, 0)),
    ] + [full(a) for a in main_args[4:]]
    agg_b = pl.pallas_call(
        functools.partial(_main_b_kernel, bt=bt, m_coarse=m_coarse),
        grid=(nb,),
        in_specs=main_b_in_specs,
        out_specs=pl.BlockSpec((bt, m_coarse, out), lambda i: (0, 0, 0)),
        out_shape=jax.ShapeDtypeStruct((bt, m_coarse, out), jnp.float32),
        scratch_shapes=[pltpu.VMEM((bt, m_coarse, out), jnp.float32)],
    )(*main_args)

    hc = pl.pallas_call(
        functools.partial(_finish_kernel, bt=bt, m_coarse=m_coarse),
        out_shape=jax.ShapeDtypeStruct((bt * m_coarse, 1, out), jnp.float32),
    )(hdst, parts, agg_b, p["upd1"]["w"], b2(p["upd1"]["b"]),
      p["upd2"]["w"], b2(p["upd2"]["b"]))

    return hc.reshape(b, t, m_coarse, out), latent_edge_index, enc


# speedup vs baseline: 1.2730x; 1.0476x over previous
import functools

import jax
import jax.numpy as jnp
from jax import lax
from jax.experimental import pallas as pl
from jax.experimental.pallas import tpu as pltpu
from jax.experimental.pallas import tpu_sc as plsc

N_FINE_TILE = 2592


def _relu(v):
    return jnp.maximum(v, 0.0)


def _lrelu(v):
    return jnp.where(v >= 0, v, 0.01 * v)


def _f32dot(a, b):
    return jnp.dot(a, b, preferred_element_type=jnp.float32)


def _prep_kernel(h3_ref, lea_ref, ei_ref,
                 dst1_w, dst1_b, dst2_w, dst2_b,
                 src2_w, src2_b, edg2_w, edg2_b,
                 msg1_w, msg1_b, msg2_w,
                 lat1_w, lat1_b, lat2_w, lat2_b,
                 lato_w, lato_b, lsk_w, lsk_b,
                 A_o, E2_o, M2_o, cb_o, hdp_o, hdst_o, enc_o,
                 dst3_o, dstc3_o, idx_o, *, n_tiles, tile, m_coarse):
    for k in range(n_tiles):
        row = ei_ref[1, pl.ds(k * tile, tile)].reshape(1, tile)
        dst3_o[k] = row
        dstc3_o[k] = jnp.transpose(row)
    n_sc = idx_o.shape[1] // 2
    dstrow = ei_ref[1:2, pl.ds(0, n_sc)]
    idx_o[0:1, 0:n_sc] = dstrow
    idx_o[0:1, n_sc:2 * n_sc] = dstrow + m_coarse
    ws = msg1_w[0:128, :]
    wd = msg1_w[128:256, :]
    we = msg1_w[256:384, :]
    A_o[...] = _f32dot(src2_w[...], ws).astype(jnp.bfloat16)
    E2_o[...] = _f32dot(edg2_w[...], we).astype(jnp.bfloat16)
    M2_o[...] = msg2_w[...].astype(jnp.bfloat16)
    cb_o[...] = (_f32dot(src2_b[...], ws) + _f32dot(edg2_b[...], we)
                 + msg1_b[...])
    hd1 = _relu(_f32dot(h3_ref[...], dst1_w[...]) + dst1_b[...])
    hdst = _f32dot(hd1, dst2_w[...]) + dst2_b[...]
    hdst_o[...] = hdst
    hdp_o[...] = _f32dot(hdst, wd).astype(jnp.bfloat16)
    lea = lea_ref[...]
    a1 = _lrelu(lea[:, 0:1] * lat1_w[0:1, :] + lea[:, 1:2] * lat1_w[1:2, :]
                + lat1_b[...])
    a2 = _lrelu(_f32dot(a1, lat2_w[...]) + lat2_b[...])
    enc_o[...] = (_f32dot(a2, lato_w[...]) + lato_b[...]
                  + lea[:, 0:1] * lsk_w[0:1, :] + lea[:, 1:2] * lsk_w[1:2, :]
                  + lsk_b[...])


def _repack_kernel(xv_ref, xstd_ref, *, bt, f):
    for b in range(bt):
        xstd_ref[b] = jnp.swapaxes(
            xv_ref[b * f:(b + 1) * f, 0, :], 0, 1).astype(jnp.bfloat16)


def _main_a_kernel(x_ref, ea_ref, dst_ref, dstc_ref,
                   src1_w, src1_b, edg1_w, edg1_b, msg2_b,
                   A_r, E2_r, M2_r, cb_r, hdp_r,
                   m_out, *, bt, m_coarse):
    tile = dst_ref.shape[2]
    dstc = dstc_ref[0]
    cols = lax.broadcasted_iota(jnp.int32, (tile, m_coarse), 1)
    oh = (cols == dstc).astype(jnp.bfloat16)

    g = _f32dot(oh, hdp_r[...])
    ea = ea_ref[...]
    e1 = _relu(ea[:, 0:1] * edg1_w[0:1, :] + ea[:, 1:2] * edg1_w[1:2, :]
               + edg1_b[...])
    epre = _f32dot(e1.astype(jnp.bfloat16), E2_r[...])
    base = epre + g + cb_r[...]
    w1 = src1_w[...].astype(jnp.bfloat16)
    for b in range(bt):
        h1 = _relu(_f32dot(x_ref[b], w1) + src1_b[...])
        hidden = _relu(_f32dot(h1.astype(jnp.bfloat16), A_r[...]) + base)
        m_out[b] = _f32dot(hidden.astype(jnp.bfloat16), M2_r[...]) + msg2_b[...]


def _main_b_kernel(x_ref, ea_ref, dst_ref, dstc_ref,
                   src1_w, src1_b, edg1_w, edg1_b, msg2_b,
                   A_r, E2_r, M2_r, cb_r, hdp_r,
                   agg_out, agg_s, *, bt, m_coarse):
    i = pl.program_id(0)

    @pl.when(i == 0)
    def _init():
        agg_s[...] = jnp.zeros_like(agg_s)

    tile = dst_ref.shape[2]
    dstt = dst_ref[0]
    rows = lax.broadcasted_iota(jnp.int32, (m_coarse, tile), 0)
    ohT = (rows == dstt).astype(jnp.bfloat16)
    dstc = dstc_ref[0]
    cols = lax.broadcasted_iota(jnp.int32, (tile, m_coarse), 1)
    oh = (cols == dstc).astype(jnp.bfloat16)

    g = _f32dot(oh, hdp_r[...])
    ea = ea_ref[...]
    e1 = _relu(ea[:, 0:1] * edg1_w[0:1, :] + ea[:, 1:2] * edg1_w[1:2, :]
               + edg1_b[...])
    epre = _f32dot(e1.astype(jnp.bfloat16), E2_r[...])
    base = epre + g + cb_r[...]
    w1 = src1_w[...].astype(jnp.bfloat16)
    for b in range(bt):
        h1 = _relu(_f32dot(x_ref[b], w1) + src1_b[...])
        hidden = _relu(_f32dot(h1.astype(jnp.bfloat16), A_r[...]) + base)
        m = (_f32dot(hidden.astype(jnp.bfloat16), M2_r[...])
             + msg2_b[...]).astype(jnp.bfloat16)
        agg_s[b] += _f32dot(ohT, m)

    @pl.when(i == pl.num_programs(0) - 1)
    def _flush():
        agg_out[...] = agg_s[...]


def _sc_scatter_kernel(m_r, idx_r, z_r, zi_r, out_r, buf_v, idx_v, acc_sh,
                       sem_m, sem_i, *, ch, full_ch, nch_per_w, tail_rows):
    cid = lax.axis_index("c")
    sid = lax.axis_index("s")
    wid = sid * 2 + cid

    @pl.when(sid == 0)
    def _init():
        pltpu.sync_copy(z_r, acc_sh)
    plsc.subcore_barrier()

    def body(j, carry):
        g = j * 32 + wid

        @pl.when(g < full_ch)
        def _full():
            base = g * ch
            cm = pltpu.async_copy(m_r.at[pl.ds(base, ch)], buf_v, sem_m)
            ci = pltpu.async_copy(idx_r.at[0, pl.ds(base, ch)], idx_v, sem_i)
            cm.wait()
            ci.wait()
            pltpu.sync_copy(buf_v, acc_sh.at[idx_v], add=True)

        if tail_rows:
            @pl.when(g == full_ch)
            def _tail():
                base = g * ch
                pad = ch - tail_rows
                pltpu.sync_copy(m_r.at[pl.ds(base, tail_rows)],
                                buf_v.at[pl.ds(0, tail_rows)])
                pltpu.sync_copy(z_r.at[pl.ds(0, pad)],
                                buf_v.at[pl.ds(tail_rows, pad)])
                pltpu.sync_copy(idx_r.at[0, pl.ds(base, tail_rows)],
                                idx_v.at[pl.ds(0, tail_rows)])
                pltpu.sync_copy(zi_r.at[0, pl.ds(0, pad)],
                                idx_v.at[pl.ds(tail_rows, pad)])
                pltpu.sync_copy(buf_v, acc_sh.at[idx_v], add=True)
        return carry

    lax.fori_loop(0, nch_per_w, body, 0)
    plsc.subcore_barrier()

    @pl.when(sid == 0)
    def _flush():
        pltpu.sync_copy(acc_sh, out_r.at[cid])


def _finish_kernel(hdst_r, part_r, aggb_r, upd1_w, upd1_b, upd2_w, upd2_b,
                   hc_out, *, bt, m_coarse):
    u1h = upd1_w[0:128, :]
    u1a = upd1_w[128:256, :]
    hpre = _f32dot(hdst_r[...], u1h) + upd1_b[...]
    for b in range(bt):
        agg = (part_r[0, pl.ds(b * m_coarse, m_coarse), :]
               + part_r[1, pl.ds(b * m_coarse, m_coarse), :]
               + aggb_r[b])
        u = _relu(hpre + _f32dot(agg, u1a))
        hc_out[pl.ds(b * m_coarse, m_coarse), 0, :] = (
            _f32dot(u, upd2_w[...]) + upd2_b[...])


def kernel(x, h3_nodes, edge_attr_f2c, latent_edge_attr, params,
           edge_index_f2c, latent_edge_index):
    b, t, n, f = x.shape
    bt = b * t
    m_coarse = h3_nodes.shape[0]
    n_lat = latent_edge_attr.shape[0]
    hid = params["src1"]["w"].shape[1]
    out = params["src2"]["w"].shape[1]
    eout = params["edg2"]["w"].shape[1]

    tile = N_FINE_TILE if n % N_FINE_TILE == 0 else max(
        d for d in range(8, 4097, 8) if n % d == 0)
    n_tiles = n // tile
    K_SC = max(1, (2 * n_tiles + 1) // 3)

    xv = jnp.transpose(x, (0, 1, 3, 2)).reshape(bt * f, 1, n)
    cn = 4096
    x2 = pl.pallas_call(
        functools.partial(_repack_kernel, bt=bt, f=f),
        grid=(pl.cdiv(n, cn),),
        in_specs=[pl.BlockSpec((bt * f, 1, cn), lambda i: (0, 0, i))],
        out_specs=pl.BlockSpec((bt, cn, f), lambda i: (0, i, 0)),
        out_shape=jax.ShapeDtypeStruct((bt, n, f), jnp.bfloat16),
    )(xv)

    def b2(v):
        return v.reshape(1, -1)

    p = params
    full = lambda a: pl.BlockSpec(a.shape, lambda i: (0,) * a.ndim)

    prep_args = (
        h3_nodes, latent_edge_attr, edge_index_f2c.astype(jnp.int32),
        p["dst1"]["w"], b2(p["dst1"]["b"]), p["dst2"]["w"], b2(p["dst2"]["b"]),
        p["src2"]["w"], b2(p["src2"]["b"]), p["edg2"]["w"], b2(p["edg2"]["b"]),
        p["msg1"]["w"], b2(p["msg1"]["b"]), p["msg2"]["w"],
        p["lat1"]["w"], b2(p["lat1"]["b"]), p["lat2"]["w"], b2(p["lat2"]["b"]),
        p["lato"]["w"], b2(p["lato"]["b"]),
        p["latskip"]["w"], b2(p["latskip"]["b"]),
    )
    prep_out = (
        jax.ShapeDtypeStruct((hid, 128), jnp.bfloat16),
        jax.ShapeDtypeStruct((hid, 128), jnp.bfloat16),
        jax.ShapeDtypeStruct((hid, 128), jnp.bfloat16),
        jax.ShapeDtypeStruct((1, 128), jnp.float32),
        jax.ShapeDtypeStruct((m_coarse, 128), jnp.bfloat16),
        jax.ShapeDtypeStruct((m_coarse, out), jnp.float32),
        jax.ShapeDtypeStruct((n_lat, eout), jnp.float32),
        jax.ShapeDtypeStruct((n_tiles, 1, tile), jnp.int32),
        jax.ShapeDtypeStruct((n_tiles, tile, 1), jnp.int32),
        jax.ShapeDtypeStruct((1, bt * K_SC * tile), jnp.int32),
    )
    (A16, E216, M216, cb, hdp16, hdst, enc, dst, dstc, idxf) = pl.pallas_call(
        functools.partial(_prep_kernel, n_tiles=n_tiles, tile=tile,
                          m_coarse=m_coarse),
        out_shape=prep_out,
    )(*prep_args)

    main_args = (
        x2, edge_attr_f2c, dst, dstc,
        p["src1"]["w"], b2(p["src1"]["b"]),
        p["edg1"]["w"], b2(p["edg1"]["b"]), b2(p["msg2"]["b"]),
        A16, E216, M216, cb, hdp16,
    )
    main_in_specs = [
        pl.BlockSpec((bt, tile, f), lambda i: (0, i, 0)),
        pl.BlockSpec((tile, 2), lambda i: (i, 0)),
        pl.BlockSpec((1, 1, tile), lambda i: (i, 0, 0)),
        pl.BlockSpec((1, tile, 1), lambda i: (i, 0, 0)),
    ] + [full(a) for a in main_args[4:]]
    m_msgs = pl.pallas_call(
        functools.partial(_main_a_kernel, bt=bt, m_coarse=m_coarse),
        grid=(K_SC,),
        in_specs=main_in_specs,
        out_specs=pl.BlockSpec((bt, tile, out), lambda i: (0, i, 0)),
        out_shape=jax.ShapeDtypeStruct((bt, K_SC * tile, out), jnp.float32),
    )(*main_args)
    m_flat = m_msgs.reshape(bt * K_SC * tile, out)

    n_rows = bt * K_SC * tile
    ch = 128
    full_ch = n_rows // ch
    tail_rows = n_rows - full_ch * ch
    nch_per_w = (full_ch + (1 if tail_rows else 0) + 31) // 32
    zeros = jnp.zeros((bt * m_coarse, out), jnp.float32)
    zeros_i = jnp.zeros((1, 128), jnp.int32)
    mesh = plsc.VectorSubcoreMesh(core_axis_name="c", subcore_axis_name="s")
    parts = pl.kernel(
        functools.partial(_sc_scatter_kernel, ch=ch, full_ch=full_ch,
                          nch_per_w=nch_per_w, tail_rows=tail_rows),
        mesh=mesh,
        out_type=jax.ShapeDtypeStruct((2, bt * m_coarse, out), jnp.float32),
        scratch_types=[
            pltpu.VMEM((ch, out), jnp.float32),
            pltpu.VMEM((ch,), jnp.int32),
            pltpu.VMEM_SHARED((bt * m_coarse, out), jnp.float32),
            pltpu.SemaphoreType.DMA,
            pltpu.SemaphoreType.DMA,
        ],
    )(m_flat, idxf, zeros, zeros_i)

    nb = n_tiles - K_SC
    main_b_in_specs = [
        pl.BlockSpec((bt, tile, f), lambda i: (0, i + K_SC, 0)),
        pl.BlockSpec((tile, 2), lambda i: (i + K_SC, 0)),
        pl.BlockSpec((1, 1, tile), lambda i: (i + K_SC, 0, 0)),
        pl.BlockSpec((1, tile, 1), lambda i: (i + K_SC, 0, 0)),
    ] + [full(a) for a in main_args[4:]]
    agg_b = pl.pallas_call(
        functools.partial(_main_b_kernel, bt=bt, m_coarse=m_coarse),
        grid=(nb,),
        in_specs=main_b_in_specs,
        out_specs=pl.BlockSpec((bt, m_coarse, out), lambda i: (0, 0, 0)),
        out_shape=jax.ShapeDtypeStruct((bt, m_coarse, out), jnp.float32),
        scratch_shapes=[pltpu.VMEM((bt, m_coarse, out), jnp.float32)],
    )(*main_args)

    hc = pl.pallas_call(
        functools.partial(_finish_kernel, bt=bt, m_coarse=m_coarse),
        out_shape=jax.ShapeDtypeStruct((bt * m_coarse, 1, out), jnp.float32),
    )(hdst, parts, agg_b, p["upd1"]["w"], b2(p["upd1"]["b"]),
      p["upd2"]["w"], b2(p["upd2"]["b"]))

    return hc.reshape(b, t, m_coarse, out), latent_edge_index, enc
